# baseline (device time: 18808 ns/iter reference)
import os

import jax
import jax.numpy as jnp
from jax import lax
from jax.experimental import pallas as pl
from jax.experimental.pallas import tpu as pltpu

_PROBE = os.environ.get("KERNEL_PROBE", "")


def kernel(Q, K, V):
    b, s, h, d = Q.shape
    hd = h * d
    da = 2 * d
    scale = d ** -0.5
    nc = 2 * b
    cs = s // 2

    def body(
        q_ref, k_ref, v_ref, out_ref,
        q_bf, k_parts, v_parts, acc,
        p1s, p1r, p2s, p2r,
    ):
        my_x = lax.axis_index("x")
        my_y = lax.axis_index("y")
        ynbr = (my_x, 1 - my_y)
        xnbr = (1 - my_x, my_y)
        role_k = my_x == 0

        k_parts[0] = (k_ref[...] * scale).astype(jnp.bfloat16)
        v_parts[0] = v_ref[...].astype(jnp.bfloat16)
        q_bf[...] = q_ref[...].astype(jnp.bfloat16)

        if _PROBE == "base":
            out_ref[...] = q_ref[...]
            return

        barrier_sem = pltpu.get_barrier_semaphore()
        for nb in (ynbr, xnbr):
            pl.semaphore_signal(
                barrier_sem, inc=1, device_id=nb,
                device_id_type=pl.DeviceIdType.MESH,
            )
        pl.semaphore_wait(barrier_sem, 2)

        def p1_desc(T, c):
            bi, off = c // 2, (c % 2) * cs
            return pltpu.make_async_remote_copy(
                src_ref=T.at[0, bi, pl.ds(off, cs)],
                dst_ref=T.at[1, bi, pl.ds(off, cs)],
                send_sem=p1s.at[c], recv_sem=p1r.at[c],
                device_id=ynbr, device_id_type=pl.DeviceIdType.MESH,
            )

        def p2_desc(T, c):
            bi, off = c // 2, (c % 2) * cs
            return pltpu.make_async_remote_copy(
                src_ref=T.at[1, bi, pl.ds(off, cs)],
                dst_ref=T.at[1, bi, pl.ds(off, cs)],
                send_sem=p2s.at[c], recv_sem=p2r.at[c],
                device_id=xnbr, device_id_type=pl.DeviceIdType.MESH,
            )

        if _PROBE == "compute":
            k_parts[1] = k_parts[0]
            v_parts[1] = v_parts[0]
        else:
            @pl.when(role_k)
            def _():
                for c in range(nc):
                    p1_desc(k_parts, c).start()

            @pl.when(jnp.logical_not(role_k))
            def _():
                for c in range(nc):
                    p1_desc(v_parts, c).start()

        ones = jnp.ones((s, d), jnp.bfloat16)

        def part_acc(part, bi, hi):
            sl = slice(hi * d, (hi + 1) * d)
            q = q_bf[bi, :, sl]
            kp = k_parts[part, bi, :, sl]
            sc = lax.dot_general(
                q, kp, (((1,), (1,)), ((), ())),
                preferred_element_type=jnp.float32,
            )
            p = jnp.exp(sc.astype(jnp.bfloat16))
            va = jnp.concatenate(
                [v_parts[part, bi, :, sl], ones], axis=1
            )
            return lax.dot_general(
                p, va, (((1,), (0,)), ((), ())),
                preferred_element_type=jnp.float32,
            )

        if _PROBE != "comm":
            for bi in range(b):
                for hi in range(h):
                    acc[bi, :, hi * da:(hi + 1) * da] = part_acc(0, bi, hi)

        if _PROBE != "compute":
            @pl.when(role_k)
            def _():
                for c in range(nc):
                    p1_desc(k_parts, c).wait_recv()
                    p2_desc(k_parts, c).start()

            @pl.when(jnp.logical_not(role_k))
            def _():
                for c in range(nc):
                    p1_desc(v_parts, c).wait_recv()
                    p2_desc(v_parts, c).start()

        for bi in range(b):
            if _PROBE not in ("comm", "compute"):
                @pl.when(role_k)
                def _(bi=bi):
                    for c in (2 * bi, 2 * bi + 1):
                        p2_desc(v_parts, c).wait_recv()

                @pl.when(jnp.logical_not(role_k))
                def _(bi=bi):
                    for c in (2 * bi, 2 * bi + 1):
                        p2_desc(k_parts, c).wait_recv()

            if _PROBE != "comm":
                for hi in range(h):
                    a = acc[bi, :, hi * da:(hi + 1) * da] + part_acc(1, bi, hi)
                    r = 1.0 / a[:, d:d + 1]
                    out_ref[bi, :, hi * d:(hi + 1) * d] = a[:, :d] * r

        if _PROBE == "comm":
            out_ref[...] = q_ref[...]
            for c in range(nc):
                @pl.when(role_k)
                def _(c=c):
                    p2_desc(v_parts, c).wait_recv()

                @pl.when(jnp.logical_not(role_k))
                def _(c=c):
                    p2_desc(k_parts, c).wait_recv()

        if _PROBE != "compute":
            @pl.when(role_k)
            def _():
                for c in range(nc):
                    p1_desc(k_parts, c).wait_send()
                    p2_desc(k_parts, c).wait_send()

            @pl.when(jnp.logical_not(role_k))
            def _():
                for c in range(nc):
                    p1_desc(v_parts, c).wait_send()
                    p2_desc(v_parts, c).wait_send()

    out = pl.pallas_call(
        body,
        out_shape=jax.ShapeDtypeStruct((b, s, hd), jnp.float32),
        in_specs=[
            pl.BlockSpec(memory_space=pltpu.VMEM),
            pl.BlockSpec(memory_space=pltpu.VMEM),
            pl.BlockSpec(memory_space=pltpu.VMEM),
        ],
        out_specs=pl.BlockSpec(memory_space=pltpu.VMEM),
        scratch_shapes=[
            pltpu.VMEM((b, s, hd), jnp.bfloat16),
            pltpu.VMEM((2, b, s, hd), jnp.bfloat16),
            pltpu.VMEM((2, b, s, hd), jnp.bfloat16),
            pltpu.VMEM((b, s, h * da), jnp.float32),
            pltpu.SemaphoreType.DMA((nc,)),
            pltpu.SemaphoreType.DMA((nc,)),
            pltpu.SemaphoreType.DMA((nc,)),
            pltpu.SemaphoreType.DMA((nc,)),
        ],
        compiler_params=pltpu.CompilerParams(collective_id=0),
    )(Q.reshape(b, s, hd), K.reshape(b, s, hd), V.reshape(b, s, hd))
    return out.reshape(b, s, h, d)


# device time: 18598 ns/iter; 1.0113x vs baseline; 1.0113x over previous
import os

import jax
import jax.numpy as jnp
from jax import lax
from jax.experimental import pallas as pl
from jax.experimental.pallas import tpu as pltpu

_PROBE = os.environ.get("KERNEL_PROBE", "")


def kernel(Q, K, V):
    b, s, h, d = Q.shape
    hd = h * d
    da = 2 * d
    scale = d ** -0.5
    nc = 2 * b
    cs = s // 2

    def body(
        q_ref, k_ref, v_ref, out_ref,
        q_bf, k_parts, v_parts, acc,
        p1s, p1r, p2s, p2r,
    ):
        my_x = lax.axis_index("x")
        my_y = lax.axis_index("y")
        ynbr = (my_x, 1 - my_y)
        xnbr = (1 - my_x, my_y)
        role_k = my_x == 0

        if _PROBE == "base":
            k_parts[0] = (k_ref[...] * scale).astype(jnp.bfloat16)
            v_parts[0] = v_ref[...].astype(jnp.bfloat16)
            q_bf[...] = q_ref[...].astype(jnp.bfloat16)
            out_ref[...] = q_ref[...].astype(out_ref.dtype)
            return

        barrier_sem = pltpu.get_barrier_semaphore()
        for nb in (ynbr, xnbr):
            pl.semaphore_signal(
                barrier_sem, inc=1, device_id=nb,
                device_id_type=pl.DeviceIdType.MESH,
            )
        pl.semaphore_wait(barrier_sem, 2)

        def p1_desc(T, c):
            bi, off = c // 2, (c % 2) * cs
            return pltpu.make_async_remote_copy(
                src_ref=T.at[0, bi, pl.ds(off, cs)],
                dst_ref=T.at[1, bi, pl.ds(off, cs)],
                send_sem=p1s.at[c], recv_sem=p1r.at[c],
                device_id=ynbr, device_id_type=pl.DeviceIdType.MESH,
            )

        def p2_desc(T, c):
            bi, off = c // 2, (c % 2) * cs
            return pltpu.make_async_remote_copy(
                src_ref=T.at[1, bi, pl.ds(off, cs)],
                dst_ref=T.at[1, bi, pl.ds(off, cs)],
                send_sem=p2s.at[c], recv_sem=p2r.at[c],
                device_id=xnbr, device_id_type=pl.DeviceIdType.MESH,
            )

        def cast_chunk(dst, src, c, mult):
            bi, off = c // 2, (c % 2) * cs
            dst[0, bi, off:off + cs] = (
                src[bi, off:off + cs, :] * mult
            ).astype(jnp.bfloat16)

        if _PROBE == "compute":
            k_parts[0] = (k_ref[...] * scale).astype(jnp.bfloat16)
            v_parts[0] = v_ref[...].astype(jnp.bfloat16)
            k_parts[1] = k_parts[0]
            v_parts[1] = v_parts[0]
        else:
            @pl.when(role_k)
            def _():
                for c in range(nc):
                    cast_chunk(k_parts, k_ref, c, scale)
                    p1_desc(k_parts, c).start()
                v_parts[0] = v_ref[...].astype(jnp.bfloat16)

            @pl.when(jnp.logical_not(role_k))
            def _():
                for c in range(nc):
                    cast_chunk(v_parts, v_ref, c, 1.0)
                    p1_desc(v_parts, c).start()
                k_parts[0] = (k_ref[...] * scale).astype(jnp.bfloat16)

        q_bf[...] = q_ref[...].astype(jnp.bfloat16)

        ones = jnp.ones((s, d), jnp.bfloat16)

        def part_acc(part, bi, hi):
            sl = slice(hi * d, (hi + 1) * d)
            q = q_bf[bi, :, sl]
            kp = k_parts[part, bi, :, sl]
            sc = lax.dot_general(
                q, kp, (((1,), (1,)), ((), ())),
                preferred_element_type=jnp.float32,
            )
            p = jnp.exp(sc.astype(jnp.bfloat16))
            va = jnp.concatenate(
                [v_parts[part, bi, :, sl], ones], axis=1
            )
            return lax.dot_general(
                p, va, (((1,), (0,)), ((), ())),
                preferred_element_type=jnp.float32,
            )

        if _PROBE != "comm":
            for bi in range(b):
                for hi in range(h):
                    acc[bi, :, hi * da:(hi + 1) * da] = part_acc(0, bi, hi)

        if _PROBE != "compute":
            @pl.when(role_k)
            def _():
                for c in range(nc):
                    p1_desc(k_parts, c).wait_recv()
                    p2_desc(k_parts, c).start()

            @pl.when(jnp.logical_not(role_k))
            def _():
                for c in range(nc):
                    p1_desc(v_parts, c).wait_recv()
                    p2_desc(v_parts, c).start()

        for bi in range(b):
            if _PROBE not in ("comm", "compute"):
                @pl.when(role_k)
                def _(bi=bi):
                    for c in (2 * bi, 2 * bi + 1):
                        p2_desc(v_parts, c).wait_recv()

                @pl.when(jnp.logical_not(role_k))
                def _(bi=bi):
                    for c in (2 * bi, 2 * bi + 1):
                        p2_desc(k_parts, c).wait_recv()

            if _PROBE != "comm":
                for hi in range(h):
                    a = acc[bi, :, hi * da:(hi + 1) * da] + part_acc(1, bi, hi)
                    r = 1.0 / a[:, d:d + 1]
                    out_ref[bi, :, hi * d:(hi + 1) * d] = (
                        a[:, :d] * r
                    ).astype(out_ref.dtype)

        if _PROBE == "comm":
            out_ref[...] = q_ref[...].astype(out_ref.dtype)
            for c in range(nc):
                @pl.when(role_k)
                def _(c=c):
                    p2_desc(v_parts, c).wait_recv()

                @pl.when(jnp.logical_not(role_k))
                def _(c=c):
                    p2_desc(k_parts, c).wait_recv()

        if _PROBE != "compute":
            @pl.when(role_k)
            def _():
                for c in range(nc):
                    p1_desc(k_parts, c).wait_send()
                    p2_desc(k_parts, c).wait_send()

            @pl.when(jnp.logical_not(role_k))
            def _():
                for c in range(nc):
                    p1_desc(v_parts, c).wait_send()
                    p2_desc(v_parts, c).wait_send()

    out = pl.pallas_call(
        body,
        out_shape=jax.ShapeDtypeStruct((b, s, hd), jnp.bfloat16),
        in_specs=[
            pl.BlockSpec(memory_space=pltpu.VMEM),
            pl.BlockSpec(memory_space=pltpu.VMEM),
            pl.BlockSpec(memory_space=pltpu.VMEM),
        ],
        out_specs=pl.BlockSpec(memory_space=pltpu.VMEM),
        scratch_shapes=[
            pltpu.VMEM((b, s, hd), jnp.bfloat16),
            pltpu.VMEM((2, b, s, hd), jnp.bfloat16),
            pltpu.VMEM((2, b, s, hd), jnp.bfloat16),
            pltpu.VMEM((b, s, h * da), jnp.float32),
            pltpu.SemaphoreType.DMA((nc,)),
            pltpu.SemaphoreType.DMA((nc,)),
            pltpu.SemaphoreType.DMA((nc,)),
            pltpu.SemaphoreType.DMA((nc,)),
        ],
        compiler_params=pltpu.CompilerParams(collective_id=0),
    )(Q.reshape(b, s, hd), K.reshape(b, s, hd), V.reshape(b, s, hd))
    return out.reshape(b, s, h, d)


# device time: 7136 ns/iter; 2.6357x vs baseline; 2.6062x over previous
import os

import jax
import jax.numpy as jnp
from jax import lax
from jax.experimental import pallas as pl
from jax.experimental.pallas import tpu as pltpu

_PROBE = os.environ.get("KERNEL_PROBE", "")


def kernel(Q, K, V):
    b, s, h, d = Q.shape
    hd = h * d
    da = 2 * d
    scale = d ** -0.5
    nc = 2 * b
    cs = s // 2

    def body(
        q_ref, k_ref, v_ref, out_ref,
        q_bf, k_parts, v_parts, acc,
        p1s, p1r, p2s, p2r,
    ):
        my_x = lax.axis_index("x")
        my_y = lax.axis_index("y")
        ynbr = (my_x, 1 - my_y)
        xnbr = (1 - my_x, my_y)
        role_k = my_x == 0

        if _PROBE == "base":
            k_parts[0] = (k_ref[...] * scale).astype(jnp.bfloat16)
            v_parts[0] = v_ref[...].astype(jnp.bfloat16)
            q_bf[...] = q_ref[...].astype(jnp.bfloat16)
            out_ref[...] = q_ref[...].astype(out_ref.dtype)
            return

        barrier_sem = pltpu.get_barrier_semaphore()
        for nb in (ynbr, xnbr):
            pl.semaphore_signal(
                barrier_sem, inc=1, device_id=nb,
                device_id_type=pl.DeviceIdType.MESH,
            )
        pl.semaphore_wait(barrier_sem, 2)

        if _PROBE == "empty":
            out_ref[...] = jnp.zeros((b, s, hd), out_ref.dtype)
            return

        def p1_desc(T, c):
            bi, off = c // 2, (c % 2) * cs
            return pltpu.make_async_remote_copy(
                src_ref=T.at[0, bi, pl.ds(off, cs)],
                dst_ref=T.at[1, bi, pl.ds(off, cs)],
                send_sem=p1s.at[c], recv_sem=p1r.at[c],
                device_id=ynbr, device_id_type=pl.DeviceIdType.MESH,
            )

        def p2_desc(T, c):
            bi, off = c // 2, (c % 2) * cs
            return pltpu.make_async_remote_copy(
                src_ref=T.at[1, bi, pl.ds(off, cs)],
                dst_ref=T.at[1, bi, pl.ds(off, cs)],
                send_sem=p2s.at[c], recv_sem=p2r.at[c],
                device_id=xnbr, device_id_type=pl.DeviceIdType.MESH,
            )

        def cast_chunk(dst, src, c, mult):
            bi, off = c // 2, (c % 2) * cs
            dst[0, bi, off:off + cs] = (
                src[bi, off:off + cs, :] * mult
            ).astype(jnp.bfloat16)

        if _PROBE == "compute":
            k_parts[0] = (k_ref[...] * scale).astype(jnp.bfloat16)
            v_parts[0] = v_ref[...].astype(jnp.bfloat16)
            k_parts[1] = k_parts[0]
            v_parts[1] = v_parts[0]
        else:
            @pl.when(role_k)
            def _():
                for c in range(nc):
                    cast_chunk(k_parts, k_ref, c, scale)
                    p1_desc(k_parts, c).start()
                v_parts[0] = v_ref[...].astype(jnp.bfloat16)

            @pl.when(jnp.logical_not(role_k))
            def _():
                for c in range(nc):
                    cast_chunk(v_parts, v_ref, c, 1.0)
                    p1_desc(v_parts, c).start()
                k_parts[0] = (k_ref[...] * scale).astype(jnp.bfloat16)

        q_bf[...] = q_ref[...].astype(jnp.bfloat16)

        ones = jnp.ones((s, d), jnp.bfloat16)

        def part_acc(part, bi, hi):
            sl = slice(hi * d, (hi + 1) * d)
            q = q_bf[bi, :, sl]
            kp = k_parts[part, bi, :, sl]
            sc = lax.dot_general(
                q, kp, (((1,), (1,)), ((), ())),
                preferred_element_type=jnp.float32,
            )
            p = jnp.exp(sc.astype(jnp.bfloat16))
            va = jnp.concatenate(
                [v_parts[part, bi, :, sl], ones], axis=1
            )
            return lax.dot_general(
                p, va, (((1,), (0,)), ((), ())),
                preferred_element_type=jnp.float32,
            )

        if _PROBE != "comm":
            for bi in range(b):
                for hi in range(h):
                    acc[bi, :, hi * da:(hi + 1) * da] = part_acc(0, bi, hi)

        if _PROBE != "compute":
            @pl.when(role_k)
            def _():
                for c in range(nc):
                    p1_desc(k_parts, c).wait_recv()
                    p2_desc(k_parts, c).start()

            @pl.when(jnp.logical_not(role_k))
            def _():
                for c in range(nc):
                    p1_desc(v_parts, c).wait_recv()
                    p2_desc(v_parts, c).start()

        for bi in range(b):
            if _PROBE not in ("comm", "compute"):
                @pl.when(role_k)
                def _(bi=bi):
                    for c in (2 * bi, 2 * bi + 1):
                        p2_desc(v_parts, c).wait_recv()

                @pl.when(jnp.logical_not(role_k))
                def _(bi=bi):
                    for c in (2 * bi, 2 * bi + 1):
                        p2_desc(k_parts, c).wait_recv()

            if _PROBE != "comm":
                for hi in range(h):
                    a = acc[bi, :, hi * da:(hi + 1) * da] + part_acc(1, bi, hi)
                    r = 1.0 / a[:, d:d + 1]
                    out_ref[bi, :, hi * d:(hi + 1) * d] = (
                        a[:, :d] * r
                    ).astype(out_ref.dtype)

        if _PROBE == "comm":
            out_ref[...] = q_ref[...].astype(out_ref.dtype)
            for c in range(nc):
                @pl.when(role_k)
                def _(c=c):
                    p2_desc(v_parts, c).wait_recv()

                @pl.when(jnp.logical_not(role_k))
                def _(c=c):
                    p2_desc(k_parts, c).wait_recv()

        if _PROBE != "compute":
            @pl.when(role_k)
            def _():
                for c in range(nc):
                    p1_desc(k_parts, c).wait_send()
                    p2_desc(k_parts, c).wait_send()

            @pl.when(jnp.logical_not(role_k))
            def _():
                for c in range(nc):
                    p1_desc(v_parts, c).wait_send()
                    p2_desc(v_parts, c).wait_send()

    out = pl.pallas_call(
        body,
        out_shape=jax.ShapeDtypeStruct((b, s, hd), jnp.bfloat16),
        in_specs=[
            pl.BlockSpec(memory_space=pltpu.VMEM),
            pl.BlockSpec(memory_space=pltpu.VMEM),
            pl.BlockSpec(memory_space=pltpu.VMEM),
        ],
        out_specs=pl.BlockSpec(memory_space=pltpu.VMEM),
        scratch_shapes=[
            pltpu.VMEM((b, s, hd), jnp.bfloat16),
            pltpu.VMEM((2, b, s, hd), jnp.bfloat16),
            pltpu.VMEM((2, b, s, hd), jnp.bfloat16),
            pltpu.VMEM((b, s, h * da), jnp.float32),
            pltpu.SemaphoreType.DMA((nc,)),
            pltpu.SemaphoreType.DMA((nc,)),
            pltpu.SemaphoreType.DMA((nc,)),
            pltpu.SemaphoreType.DMA((nc,)),
        ],
        compiler_params=pltpu.CompilerParams(collective_id=0),
    )(Q.reshape(b, s, hd), K.reshape(b, s, hd), V.reshape(b, s, hd))
    return out.reshape(b, s, h, d)


# device time: 5308 ns/iter; 3.5433x vs baseline; 1.3444x over previous
import os

import jax
import jax.numpy as jnp
from jax import lax
from jax.experimental import pallas as pl
from jax.experimental.pallas import tpu as pltpu

_PROBE = os.environ.get("KERNEL_PROBE", "")


def kernel(Q, K, V):
    b, s, h, d = Q.shape
    hd = h * d
    da = 2 * d
    scale = d ** -0.5
    nc = 2 * b
    cs = s // 2

    if _PROBE == "empty2":
        def body2(q_ref, out_ref):
            my_x = lax.axis_index("x")
            my_y = lax.axis_index("y")
            barrier_sem = pltpu.get_barrier_semaphore()
            for nb in ((my_x, 1 - my_y), (1 - my_x, my_y)):
                pl.semaphore_signal(
                    barrier_sem, inc=1, device_id=nb,
                    device_id_type=pl.DeviceIdType.MESH,
                )
            pl.semaphore_wait(barrier_sem, 2)
            out_ref[...] = jnp.zeros((b, s, hd), out_ref.dtype)

        out = pl.pallas_call(
            body2,
            out_shape=jax.ShapeDtypeStruct((b, s, hd), jnp.bfloat16),
            in_specs=[pl.BlockSpec(memory_space=pltpu.VMEM)],
            out_specs=pl.BlockSpec(memory_space=pltpu.VMEM),
            compiler_params=pltpu.CompilerParams(collective_id=0),
        )(Q.reshape(b, s, hd))
        return out.reshape(b, s, h, d)

    def body(
        q_ref, k_ref, v_ref, out_ref,
        q_bf, k_parts, v_parts, acc,
        p1s, p1r, p2s, p2r,
    ):
        my_x = lax.axis_index("x")
        my_y = lax.axis_index("y")
        ynbr = (my_x, 1 - my_y)
        xnbr = (1 - my_x, my_y)
        role_k = my_x == 0

        if _PROBE == "base":
            k_parts[0] = (k_ref[...] * scale).astype(jnp.bfloat16)
            v_parts[0] = v_ref[...].astype(jnp.bfloat16)
            q_bf[...] = q_ref[...].astype(jnp.bfloat16)
            out_ref[...] = q_ref[...].astype(out_ref.dtype)
            return

        barrier_sem = pltpu.get_barrier_semaphore()
        for nb in (ynbr, xnbr):
            pl.semaphore_signal(
                barrier_sem, inc=1, device_id=nb,
                device_id_type=pl.DeviceIdType.MESH,
            )
        pl.semaphore_wait(barrier_sem, 2)

        if _PROBE == "empty":
            out_ref[...] = jnp.zeros((b, s, hd), out_ref.dtype)
            return

        def p1_desc(T, c):
            bi, off = c // 2, (c % 2) * cs
            return pltpu.make_async_remote_copy(
                src_ref=T.at[0, bi, pl.ds(off, cs)],
                dst_ref=T.at[1, bi, pl.ds(off, cs)],
                send_sem=p1s.at[c], recv_sem=p1r.at[c],
                device_id=ynbr, device_id_type=pl.DeviceIdType.MESH,
            )

        def p2_desc(T, c):
            bi, off = c // 2, (c % 2) * cs
            return pltpu.make_async_remote_copy(
                src_ref=T.at[1, bi, pl.ds(off, cs)],
                dst_ref=T.at[1, bi, pl.ds(off, cs)],
                send_sem=p2s.at[c], recv_sem=p2r.at[c],
                device_id=xnbr, device_id_type=pl.DeviceIdType.MESH,
            )

        def cast_chunk(dst, src, c, mult):
            bi, off = c // 2, (c % 2) * cs
            dst[0, bi, off:off + cs] = (
                src[bi, off:off + cs, :] * mult
            ).astype(jnp.bfloat16)

        if _PROBE == "compute":
            k_parts[0] = (k_ref[...] * scale).astype(jnp.bfloat16)
            v_parts[0] = v_ref[...].astype(jnp.bfloat16)
            k_parts[1] = k_parts[0]
            v_parts[1] = v_parts[0]
        else:
            @pl.when(role_k)
            def _():
                for c in range(nc):
                    cast_chunk(k_parts, k_ref, c, scale)
                    p1_desc(k_parts, c).start()
                v_parts[0] = v_ref[...].astype(jnp.bfloat16)

            @pl.when(jnp.logical_not(role_k))
            def _():
                for c in range(nc):
                    cast_chunk(v_parts, v_ref, c, 1.0)
                    p1_desc(v_parts, c).start()
                k_parts[0] = (k_ref[...] * scale).astype(jnp.bfloat16)

        q_bf[...] = q_ref[...].astype(jnp.bfloat16)

        ones = jnp.ones((s, d), jnp.bfloat16)

        def part_acc(part, bi, hi):
            sl = slice(hi * d, (hi + 1) * d)
            q = q_bf[bi, :, sl]
            kp = k_parts[part, bi, :, sl]
            sc = lax.dot_general(
                q, kp, (((1,), (1,)), ((), ())),
                preferred_element_type=jnp.float32,
            )
            p = jnp.exp(sc.astype(jnp.bfloat16))
            va = jnp.concatenate(
                [v_parts[part, bi, :, sl], ones], axis=1
            )
            return lax.dot_general(
                p, va, (((1,), (0,)), ((), ())),
                preferred_element_type=jnp.float32,
            )

        if _PROBE != "comm":
            for bi in range(b):
                for hi in range(h):
                    acc[bi, :, hi * da:(hi + 1) * da] = part_acc(0, bi, hi)

        if _PROBE != "compute":
            @pl.when(role_k)
            def _():
                for c in range(nc):
                    p1_desc(k_parts, c).wait_recv()
                    p2_desc(k_parts, c).start()

            @pl.when(jnp.logical_not(role_k))
            def _():
                for c in range(nc):
                    p1_desc(v_parts, c).wait_recv()
                    p2_desc(v_parts, c).start()

        for bi in range(b):
            if _PROBE not in ("comm", "compute"):
                @pl.when(role_k)
                def _(bi=bi):
                    for c in (2 * bi, 2 * bi + 1):
                        p2_desc(v_parts, c).wait_recv()

                @pl.when(jnp.logical_not(role_k))
                def _(bi=bi):
                    for c in (2 * bi, 2 * bi + 1):
                        p2_desc(k_parts, c).wait_recv()

            if _PROBE != "comm":
                for hi in range(h):
                    a = acc[bi, :, hi * da:(hi + 1) * da] + part_acc(1, bi, hi)
                    r = 1.0 / a[:, d:d + 1]
                    out_ref[bi, :, hi * d:(hi + 1) * d] = (
                        a[:, :d] * r
                    ).astype(out_ref.dtype)

        if _PROBE == "comm":
            out_ref[...] = q_ref[...].astype(out_ref.dtype)
            for c in range(nc):
                @pl.when(role_k)
                def _(c=c):
                    p2_desc(v_parts, c).wait_recv()

                @pl.when(jnp.logical_not(role_k))
                def _(c=c):
                    p2_desc(k_parts, c).wait_recv()

        if _PROBE != "compute":
            @pl.when(role_k)
            def _():
                for c in range(nc):
                    p1_desc(k_parts, c).wait_send()
                    p2_desc(k_parts, c).wait_send()

            @pl.when(jnp.logical_not(role_k))
            def _():
                for c in range(nc):
                    p1_desc(v_parts, c).wait_send()
                    p2_desc(v_parts, c).wait_send()

    out = pl.pallas_call(
        body,
        out_shape=jax.ShapeDtypeStruct((b, s, hd), jnp.bfloat16),
        in_specs=[
            pl.BlockSpec(memory_space=pltpu.VMEM),
            pl.BlockSpec(memory_space=pltpu.VMEM),
            pl.BlockSpec(memory_space=pltpu.VMEM),
        ],
        out_specs=pl.BlockSpec(memory_space=pltpu.VMEM),
        scratch_shapes=[
            pltpu.VMEM((b, s, hd), jnp.bfloat16),
            pltpu.VMEM((2, b, s, hd), jnp.bfloat16),
            pltpu.VMEM((2, b, s, hd), jnp.bfloat16),
            pltpu.VMEM((b, s, h * da), jnp.float32),
            pltpu.SemaphoreType.DMA((nc,)),
            pltpu.SemaphoreType.DMA((nc,)),
            pltpu.SemaphoreType.DMA((nc,)),
            pltpu.SemaphoreType.DMA((nc,)),
        ],
        compiler_params=pltpu.CompilerParams(collective_id=0),
    )(Q.reshape(b, s, hd), K.reshape(b, s, hd), V.reshape(b, s, hd))
    return out.reshape(b, s, h, d)
